# Initial kernel scaffold; baseline (speedup 1.0000x reference)
import jax
import jax.numpy as jnp
from jax.experimental import pallas as pl


def kernel(xyz, points, new_xyz, nn_idx, w0, b0, g0, be0, w1, b1, g1, be1, w2, b2, g2, be2, wl, bl, gl, bel):
    raise NotImplementedError("write your pallas kernel here")


# speedup vs baseline: 8.2764x; 8.2764x over previous
import functools

import jax
import jax.numpy as jnp
from jax import lax
from jax.experimental import pallas as pl
from jax.experimental.pallas import tpu as pltpu
from jax.experimental.pallas import tpu_sc as plsc

WIDTH = 80
SBLK = 256
K = 16
CMID = 16
CPAD = 72


def _sc_gather(tbl, idx, n_rows):
    info = plsc.get_sparse_core_info()
    nw = info.num_cores * info.num_subcores
    per_w = n_rows // nw
    ch = 1024
    n_ch = per_w // ch
    mesh = plsc.VectorSubcoreMesh(core_axis_name="c", subcore_axis_name="s")

    @functools.partial(
        pl.kernel,
        mesh=mesh,
        compiler_params=pltpu.CompilerParams(use_tc_tiling_on_sc=False),
        out_type=jax.ShapeDtypeStruct((n_rows, WIDTH), jnp.float32),
        scratch_types=[
            pltpu.VMEM((ch,), jnp.int32),
            pltpu.VMEM((ch, WIDTH), jnp.float32),
            pltpu.SemaphoreType.DMA,
        ],
    )
    def k(tbl_hbm, idx_hbm, out_hbm, idx_v, rows_v, sem):
        wid = lax.axis_index("s") * info.num_cores + lax.axis_index("c")

        def body(i, carry):
            base = wid * per_w + i * ch
            pltpu.sync_copy(idx_hbm.at[pl.ds(base, ch)], idx_v)
            pltpu.async_copy(tbl_hbm.at[idx_v], rows_v, sem).wait()
            pltpu.sync_copy(rows_v, out_hbm.at[pl.ds(base, ch)])
            return carry

        lax.fori_loop(0, n_ch, body, 0)

    return k(tbl, idx)


def _leaky(x):
    return jnp.where(x >= 0, x, 0.2 * x)


def _ln(x, g, b, eps=1e-5):
    m = jnp.mean(x, axis=-1, keepdims=True)
    v = jnp.mean((x - m) * (x - m), axis=-1, keepdims=True)
    return (x - m) * lax.rsqrt(v + eps) * g + b


def _tc_body(g_ref, nxt_ref,
             w0_ref, b0_ref, g0_ref, be0_ref,
             w1_ref, b1_ref, g1_ref, be1_ref,
             w2_ref, b2_ref, g2_ref, be2_ref,
             wlp_ref, bl_ref, gl_ref, bel_ref,
             out_ref):
    g = g_ref[0]
    nxt = nxt_ref[0]
    rows = SBLK * K
    nx_r = jnp.broadcast_to(nxt[:, None, :], (SBLK, K, 3)).reshape(rows, 3)
    gxn = g[:, 64:67] - nx_r

    h = _leaky(_ln(gxn @ w0_ref[...] + b0_ref[...], g0_ref[...], be0_ref[...]))
    h = _leaky(_ln(h @ w1_ref[...] + b1_ref[...], g1_ref[...], be1_ref[...]))
    w = _leaky(_ln(h @ w2_ref[...] + b2_ref[...], g2_ref[...], be2_ref[...]))

    a = jnp.concatenate(
        [g[:, :64], gxn, jnp.zeros((rows, CPAD - 67), jnp.float32)], axis=-1)
    a3 = a.reshape(SBLK, K, CPAD)
    w3 = w.reshape(SBLK, K, CMID)

    parts = [jnp.sum(a3 * w3[:, :, m][:, :, None], axis=1) for m in range(CMID)]
    outf = jnp.concatenate(parts, axis=-1)

    f = outf @ wlp_ref[...] + bl_ref[...]
    f = _leaky(_ln(f, gl_ref[...], bel_ref[...]))
    out_ref[0] = f


def kernel(xyz, points, new_xyz, nn_idx,
           w0, b0, g0, be0,
           w1, b1, g1, be1,
           w2, b2, g2, be2,
           wl, bl, gl, bel):
    B, C, N = xyz.shape
    _, D, _ = points.shape
    _, S, _ = nn_idx.shape
    out_ch = wl.shape[1]

    tbl = jnp.concatenate(
        [points.transpose(0, 2, 1), xyz.transpose(0, 2, 1),
         jnp.zeros((B, N, WIDTH - D - C), jnp.float32)], axis=-1
    ).reshape(B * N, WIDTH)
    flat_idx = (nn_idx.reshape(B, S * K).astype(jnp.int32)
                + (jnp.arange(B, dtype=jnp.int32) * N)[:, None]).reshape(-1)

    gathered = _sc_gather(tbl, flat_idx, B * S * K)

    nxt = new_xyz.transpose(0, 2, 1)
    wlr = wl.reshape(C + D, CMID, out_ch).transpose(1, 0, 2)
    wlp = jnp.zeros((CMID, CPAD, out_ch), jnp.float32)
    wlp = wlp.at[:, :C + D].set(wlr).reshape(CMID * CPAD, out_ch)

    grid = (B, S // SBLK)
    out = pl.pallas_call(
        _tc_body,
        grid=grid,
        in_specs=[
            pl.BlockSpec((1, SBLK * K, WIDTH), lambda b, s: (b, s, 0)),
            pl.BlockSpec((1, SBLK, 3), lambda b, s: (b, s, 0)),
            pl.BlockSpec((C, 8), lambda b, s: (0, 0)),
            pl.BlockSpec((1, 8), lambda b, s: (0, 0)),
            pl.BlockSpec((1, 8), lambda b, s: (0, 0)),
            pl.BlockSpec((1, 8), lambda b, s: (0, 0)),
            pl.BlockSpec((8, 8), lambda b, s: (0, 0)),
            pl.BlockSpec((1, 8), lambda b, s: (0, 0)),
            pl.BlockSpec((1, 8), lambda b, s: (0, 0)),
            pl.BlockSpec((1, 8), lambda b, s: (0, 0)),
            pl.BlockSpec((8, CMID), lambda b, s: (0, 0)),
            pl.BlockSpec((1, CMID), lambda b, s: (0, 0)),
            pl.BlockSpec((1, CMID), lambda b, s: (0, 0)),
            pl.BlockSpec((1, CMID), lambda b, s: (0, 0)),
            pl.BlockSpec((CMID * CPAD, out_ch), lambda b, s: (0, 0)),
            pl.BlockSpec((1, out_ch), lambda b, s: (0, 0)),
            pl.BlockSpec((1, out_ch), lambda b, s: (0, 0)),
            pl.BlockSpec((1, out_ch), lambda b, s: (0, 0)),
        ],
        out_specs=pl.BlockSpec((1, SBLK, out_ch), lambda b, s: (b, s, 0)),
        out_shape=jax.ShapeDtypeStruct((B, S, out_ch), jnp.float32),
    )(gathered.reshape(B, S * K, WIDTH), nxt,
      w0, b0.reshape(1, -1), g0.reshape(1, -1), be0.reshape(1, -1),
      w1, b1.reshape(1, -1), g1.reshape(1, -1), be1.reshape(1, -1),
      w2, b2.reshape(1, -1), g2.reshape(1, -1), be2.reshape(1, -1),
      wlp, bl.reshape(1, -1), gl.reshape(1, -1), bel.reshape(1, -1))

    return out.transpose(0, 2, 1)


# speedup vs baseline: 9.3687x; 1.1320x over previous
import functools

import jax
import jax.numpy as jnp
from jax import lax
from jax.experimental import pallas as pl
from jax.experimental.pallas import tpu as pltpu
from jax.experimental.pallas import tpu_sc as plsc

WIDTH = 80
SBLK = 256
K = 16
CMID = 16
CPAD = 72


def _sc_gather(tbl, idx, n_rows):
    info = plsc.get_sparse_core_info()
    nw = info.num_cores * info.num_subcores
    per_w = n_rows // nw
    ch = 1024
    n_ch = per_w // ch
    mesh = plsc.VectorSubcoreMesh(core_axis_name="c", subcore_axis_name="s")

    @functools.partial(
        pl.kernel,
        mesh=mesh,
        compiler_params=pltpu.CompilerParams(use_tc_tiling_on_sc=False),
        out_type=jax.ShapeDtypeStruct((n_rows, WIDTH), jnp.float32),
        scratch_types=[
            pltpu.VMEM((ch,), jnp.int32),
            pltpu.VMEM((ch, WIDTH), jnp.float32),
            pltpu.SemaphoreType.DMA,
        ],
    )
    def k(tbl_hbm, idx_hbm, out_hbm, idx_v, rows_v, sem):
        wid = lax.axis_index("s") * info.num_cores + lax.axis_index("c")

        def body(i, carry):
            base = wid * per_w + i * ch
            pltpu.sync_copy(idx_hbm.at[pl.ds(base, ch)], idx_v)
            pltpu.async_copy(tbl_hbm.at[idx_v], rows_v, sem).wait()
            pltpu.sync_copy(rows_v, out_hbm.at[pl.ds(base, ch)])
            return carry

        lax.fori_loop(0, n_ch, body, 0)

    return k(tbl, idx)


def _leaky(x):
    return jnp.where(x >= 0, x, 0.2 * x)


def _ln_post(d, g, b, eps=1e-5):
    n = d.shape[-1]
    j = jnp.full((n, n), 1.0 / n, jnp.float32)
    v = (d * d) @ j
    return d * lax.rsqrt(v + eps) * g + b


def _tc_body(g_ref, nxt_ref,
             w0_ref, b0_ref, g0_ref, be0_ref,
             w1_ref, b1_ref, g1_ref, be1_ref,
             w2_ref, b2_ref, g2_ref, be2_ref,
             wlp_ref, bl_ref, gl_ref, bel_ref,
             out_ref):
    g = g_ref[0]
    nxt = nxt_ref[0]
    rows = SBLK * K
    nx_r = jnp.broadcast_to(nxt[:, None, :], (SBLK, K, 3)).reshape(rows, 3)
    gxn = g[:, 64:67] - nx_r

    h = _leaky(_ln_post(gxn @ w0_ref[...] + b0_ref[...], g0_ref[...], be0_ref[...]))
    h = _leaky(_ln_post(h @ w1_ref[...] + b1_ref[...], g1_ref[...], be1_ref[...]))
    w = _leaky(_ln_post(h @ w2_ref[...] + b2_ref[...], g2_ref[...], be2_ref[...]))

    a = jnp.concatenate(
        [g[:, :64], gxn, jnp.zeros((rows, CPAD - 67), jnp.float32)], axis=-1)
    a3 = a.reshape(SBLK, K, CPAD)
    w3 = w.reshape(SBLK, K, CMID)

    d = bl_ref[...]
    for m in range(CMID):
        part = jnp.sum(a3 * w3[:, :, m][:, :, None], axis=1)
        d = d + part @ wlp_ref[m]
    f = _leaky(_ln_post(d, gl_ref[...], bel_ref[...]))
    out_ref[0] = f


def kernel(xyz, points, new_xyz, nn_idx,
           w0, b0, g0, be0,
           w1, b1, g1, be1,
           w2, b2, g2, be2,
           wl, bl, gl, bel):
    B, C, N = xyz.shape
    _, D, _ = points.shape
    _, S, _ = nn_idx.shape
    out_ch = wl.shape[1]

    tbl = jnp.concatenate(
        [points.transpose(0, 2, 1), xyz.transpose(0, 2, 1),
         jnp.zeros((B, N, WIDTH - D - C), jnp.float32)], axis=-1
    ).reshape(B * N, WIDTH)
    flat_idx = (nn_idx.reshape(B, S * K).astype(jnp.int32)
                + (jnp.arange(B, dtype=jnp.int32) * N)[:, None]).reshape(-1)

    gathered = _sc_gather(tbl, flat_idx, B * S * K)

    nxt = new_xyz.transpose(0, 2, 1)

    def center(wm, bv):
        n = wm.shape[1]
        cm = jnp.eye(n, dtype=jnp.float32) - jnp.full((n, n), 1.0 / n)
        return wm @ cm, bv @ cm

    w0c, b0c = center(w0, b0)
    w1c, b1c = center(w1, b1)
    w2c, b2c = center(w2, b2)
    wlr = wl.reshape(C + D, CMID, out_ch).transpose(1, 0, 2)
    wlp = jnp.zeros((CMID, CPAD, out_ch), jnp.float32).at[:, :C + D].set(wlr)
    c64 = jnp.eye(out_ch, dtype=jnp.float32) - jnp.full((out_ch, out_ch), 1.0 / out_ch)
    wlp = wlp @ c64
    blc = bl @ c64

    grid = (B, S // SBLK)
    out = pl.pallas_call(
        _tc_body,
        grid=grid,
        in_specs=[
            pl.BlockSpec((1, SBLK * K, WIDTH), lambda b, s: (b, s, 0)),
            pl.BlockSpec((1, SBLK, 3), lambda b, s: (b, s, 0)),
            pl.BlockSpec((C, 8), lambda b, s: (0, 0)),
            pl.BlockSpec((1, 8), lambda b, s: (0, 0)),
            pl.BlockSpec((1, 8), lambda b, s: (0, 0)),
            pl.BlockSpec((1, 8), lambda b, s: (0, 0)),
            pl.BlockSpec((8, 8), lambda b, s: (0, 0)),
            pl.BlockSpec((1, 8), lambda b, s: (0, 0)),
            pl.BlockSpec((1, 8), lambda b, s: (0, 0)),
            pl.BlockSpec((1, 8), lambda b, s: (0, 0)),
            pl.BlockSpec((8, CMID), lambda b, s: (0, 0)),
            pl.BlockSpec((1, CMID), lambda b, s: (0, 0)),
            pl.BlockSpec((1, CMID), lambda b, s: (0, 0)),
            pl.BlockSpec((1, CMID), lambda b, s: (0, 0)),
            pl.BlockSpec((CMID, CPAD, out_ch), lambda b, s: (0, 0, 0)),
            pl.BlockSpec((1, out_ch), lambda b, s: (0, 0)),
            pl.BlockSpec((1, out_ch), lambda b, s: (0, 0)),
            pl.BlockSpec((1, out_ch), lambda b, s: (0, 0)),
        ],
        out_specs=pl.BlockSpec((1, SBLK, out_ch), lambda b, s: (b, s, 0)),
        out_shape=jax.ShapeDtypeStruct((B, S, out_ch), jnp.float32),
    )(gathered.reshape(B, S * K, WIDTH), nxt,
      w0c, b0c.reshape(1, -1), g0.reshape(1, -1), be0.reshape(1, -1),
      w1c, b1c.reshape(1, -1), g1.reshape(1, -1), be1.reshape(1, -1),
      w2c, b2c.reshape(1, -1), g2.reshape(1, -1), be2.reshape(1, -1),
      wlp, blc.reshape(1, -1), gl.reshape(1, -1), bel.reshape(1, -1))

    return out.transpose(0, 2, 1)


# speedup vs baseline: 18.6109x; 1.9865x over previous
import functools

import jax
import jax.numpy as jnp
from jax import lax
from jax.experimental import pallas as pl
from jax.experimental.pallas import tpu as pltpu
from jax.experimental.pallas import tpu_sc as plsc

WIDTH = 80
SBLK = 256
K = 16
CMID = 16
CPAD = 72


def _sc_gather(tbl, idx, n_rows):
    info = plsc.get_sparse_core_info()
    nw = info.num_cores * info.num_subcores
    per_w = n_rows // nw
    ch = 1024
    n_ch = per_w // ch
    mesh = plsc.VectorSubcoreMesh(core_axis_name="c", subcore_axis_name="s")

    @functools.partial(
        pl.kernel,
        mesh=mesh,
        compiler_params=pltpu.CompilerParams(use_tc_tiling_on_sc=False),
        out_type=jax.ShapeDtypeStruct((n_rows, WIDTH), jnp.float32),
        scratch_types=[
            pltpu.VMEM((ch,), jnp.int32),
            pltpu.VMEM((ch, WIDTH), jnp.float32),
            pltpu.SemaphoreType.DMA,
        ],
    )
    def k(tbl_hbm, idx_hbm, out_hbm, idx_v, rows_v, sem):
        wid = lax.axis_index("s") * info.num_cores + lax.axis_index("c")

        def body(i, carry):
            base = wid * per_w + i * ch
            pltpu.sync_copy(idx_hbm.at[pl.ds(base, ch)], idx_v)
            pltpu.async_copy(tbl_hbm.at[idx_v], rows_v, sem).wait()
            pltpu.sync_copy(rows_v, out_hbm.at[pl.ds(base, ch)])
            return carry

        lax.fori_loop(0, n_ch, body, 0)

    return k(tbl, idx)


def _leaky(x):
    return jnp.where(x >= 0, x, 0.2 * x)


def _ln_post(d, g, b, eps=1e-5):
    n = d.shape[-1]
    j = jnp.full((n, n), 1.0 / n, jnp.float32)
    v = (d * d) @ j
    return d * lax.rsqrt(v + eps) * g + b


def _tc_body(g_ref, nxt_ref,
             w0_ref, b0_ref, g0_ref, be0_ref,
             w1_ref, b1_ref, g1_ref, be1_ref,
             w2_ref, b2_ref, g2_ref, be2_ref,
             wlp_ref, bl_ref, gl_ref, bel_ref,
             out_ref):
    g = g_ref[0]
    nxt = nxt_ref[0]
    rows = SBLK * K
    nx_r = jnp.broadcast_to(nxt[:, None, :], (SBLK, K, 3)).reshape(rows, 3)
    gxn = g[:, 64:67] - nx_r

    h = _leaky(_ln_post(gxn @ w0_ref[...] + b0_ref[...], g0_ref[...], be0_ref[...]))
    h = _leaky(_ln_post(h @ w1_ref[...] + b1_ref[...], g1_ref[...], be1_ref[...]))
    w = _leaky(_ln_post(h @ w2_ref[...] + b2_ref[...], g2_ref[...], be2_ref[...]))

    a = jnp.concatenate(
        [g[:, :64], gxn, jnp.zeros((rows, CPAD - 67), jnp.float32)], axis=-1)
    a3 = a.reshape(SBLK, K, CPAD)
    w3 = w.reshape(SBLK, K, CMID)

    out3 = lax.dot_general(w3, a3, (((1,), (1,)), ((0,), (0,))))
    d = bl_ref[...] + out3.reshape(SBLK, CMID * CPAD) @ wlp_ref[...].reshape(
        CMID * CPAD, 64)
    f = _leaky(_ln_post(d, gl_ref[...], bel_ref[...]))
    out_ref[0] = f


def kernel(xyz, points, new_xyz, nn_idx,
           w0, b0, g0, be0,
           w1, b1, g1, be1,
           w2, b2, g2, be2,
           wl, bl, gl, bel):
    B, C, N = xyz.shape
    _, D, _ = points.shape
    _, S, _ = nn_idx.shape
    out_ch = wl.shape[1]

    tbl = jnp.concatenate(
        [points.transpose(0, 2, 1), xyz.transpose(0, 2, 1),
         jnp.zeros((B, N, WIDTH - D - C), jnp.float32)], axis=-1
    ).reshape(B * N, WIDTH)
    flat_idx = (nn_idx.reshape(B, S * K).astype(jnp.int32)
                + (jnp.arange(B, dtype=jnp.int32) * N)[:, None]).reshape(-1)

    gathered = _sc_gather(tbl, flat_idx, B * S * K)

    nxt = new_xyz.transpose(0, 2, 1)

    def center(wm, bv):
        n = wm.shape[1]
        cm = jnp.eye(n, dtype=jnp.float32) - jnp.full((n, n), 1.0 / n)
        return wm @ cm, bv @ cm

    w0c, b0c = center(w0, b0)
    w1c, b1c = center(w1, b1)
    w2c, b2c = center(w2, b2)
    wlr = wl.reshape(C + D, CMID, out_ch).transpose(1, 0, 2)
    wlp = jnp.zeros((CMID, CPAD, out_ch), jnp.float32).at[:, :C + D].set(wlr)
    c64 = jnp.eye(out_ch, dtype=jnp.float32) - jnp.full((out_ch, out_ch), 1.0 / out_ch)
    wlp = wlp @ c64
    blc = bl @ c64

    grid = (B, S // SBLK)
    out = pl.pallas_call(
        _tc_body,
        grid=grid,
        in_specs=[
            pl.BlockSpec((1, SBLK * K, WIDTH), lambda b, s: (b, s, 0)),
            pl.BlockSpec((1, SBLK, 3), lambda b, s: (b, s, 0)),
            pl.BlockSpec((C, 8), lambda b, s: (0, 0)),
            pl.BlockSpec((1, 8), lambda b, s: (0, 0)),
            pl.BlockSpec((1, 8), lambda b, s: (0, 0)),
            pl.BlockSpec((1, 8), lambda b, s: (0, 0)),
            pl.BlockSpec((8, 8), lambda b, s: (0, 0)),
            pl.BlockSpec((1, 8), lambda b, s: (0, 0)),
            pl.BlockSpec((1, 8), lambda b, s: (0, 0)),
            pl.BlockSpec((1, 8), lambda b, s: (0, 0)),
            pl.BlockSpec((8, CMID), lambda b, s: (0, 0)),
            pl.BlockSpec((1, CMID), lambda b, s: (0, 0)),
            pl.BlockSpec((1, CMID), lambda b, s: (0, 0)),
            pl.BlockSpec((1, CMID), lambda b, s: (0, 0)),
            pl.BlockSpec((CMID, CPAD, out_ch), lambda b, s: (0, 0, 0)),
            pl.BlockSpec((1, out_ch), lambda b, s: (0, 0)),
            pl.BlockSpec((1, out_ch), lambda b, s: (0, 0)),
            pl.BlockSpec((1, out_ch), lambda b, s: (0, 0)),
        ],
        out_specs=pl.BlockSpec((1, SBLK, out_ch), lambda b, s: (b, s, 0)),
        out_shape=jax.ShapeDtypeStruct((B, S, out_ch), jnp.float32),
    )(gathered.reshape(B, S * K, WIDTH), nxt,
      w0c, b0c.reshape(1, -1), g0.reshape(1, -1), be0.reshape(1, -1),
      w1c, b1c.reshape(1, -1), g1.reshape(1, -1), be1.reshape(1, -1),
      w2c, b2c.reshape(1, -1), g2.reshape(1, -1), be2.reshape(1, -1),
      wlp, blc.reshape(1, -1), gl.reshape(1, -1), bel.reshape(1, -1))

    return out.transpose(0, 2, 1)


# speedup vs baseline: 21.3207x; 1.1456x over previous
import functools

import jax
import jax.numpy as jnp
from jax import lax
from jax.experimental import pallas as pl
from jax.experimental.pallas import tpu as pltpu
from jax.experimental.pallas import tpu_sc as plsc

WIDTH = 128
SBLK = 256
K = 16
CMID = 16
CPAD = 128


def _sc_gather(tbl, idx, n_rows):
    info = plsc.get_sparse_core_info()
    nw = info.num_cores * info.num_subcores
    per_w = n_rows // nw
    ch = 512
    n_ch = per_w // ch
    mesh = plsc.VectorSubcoreMesh(core_axis_name="c", subcore_axis_name="s")

    @functools.partial(
        pl.kernel,
        mesh=mesh,
        out_type=jax.ShapeDtypeStruct((n_rows, WIDTH), jnp.float32),
        scratch_types=[
            pltpu.VMEM((ch,), jnp.int32),
            pltpu.VMEM((ch, WIDTH), jnp.float32),
            pltpu.SemaphoreType.DMA,
        ],
    )
    def k(tbl_hbm, idx_hbm, out_hbm, idx_v, rows_v, sem):
        wid = lax.axis_index("s") * info.num_cores + lax.axis_index("c")

        def body(i, carry):
            base = wid * per_w + i * ch
            pltpu.sync_copy(idx_hbm.at[pl.ds(base, ch)], idx_v)
            pltpu.async_copy(tbl_hbm.at[idx_v], rows_v, sem).wait()
            pltpu.sync_copy(rows_v, out_hbm.at[pl.ds(base, ch)])
            return carry

        lax.fori_loop(0, n_ch, body, 0)

    return k(tbl, idx)


def _leaky(x):
    return jnp.where(x >= 0, x, 0.2 * x)


def _ln_post(d, g, b, eps=1e-5):
    n = d.shape[-1]
    j = jnp.full((n, n), 1.0 / n, jnp.float32)
    v = (d * d) @ j
    return d * lax.rsqrt(v + eps) * g + b


def _tc_body(g_ref, nxt_ref,
             w0_ref, b0_ref, g0_ref, be0_ref,
             w1_ref, b1_ref, g1_ref, be1_ref,
             w2_ref, b2_ref, g2_ref, be2_ref,
             wlp_ref, bl_ref, gl_ref, bel_ref,
             out_ref):
    g = g_ref[0]
    rows = SBLK * K
    gxn = g[:, 64:67] - nxt_ref[0]

    h = _leaky(_ln_post(gxn @ w0_ref[...] + b0_ref[...], g0_ref[...], be0_ref[...]))
    h = _leaky(_ln_post(h @ w1_ref[...] + b1_ref[...], g1_ref[...], be1_ref[...]))
    w = _leaky(_ln_post(h @ w2_ref[...] + b2_ref[...], g2_ref[...], be2_ref[...]))

    a = jnp.concatenate(
        [g[:, :64], gxn, jnp.zeros((rows, CPAD - 67), jnp.float32)], axis=-1)
    a3 = a.reshape(SBLK, K, CPAD)
    w3 = w.reshape(SBLK, K, CMID)

    out3 = lax.dot_general(w3, a3, (((1,), (1,)), ((0,), (0,))))
    d = bl_ref[...] + out3.reshape(SBLK, CMID * CPAD) @ wlp_ref[...].reshape(
        CMID * CPAD, 64)
    f = _leaky(_ln_post(d, gl_ref[...], bel_ref[...]))
    out_ref[0] = f


def kernel(xyz, points, new_xyz, nn_idx,
           w0, b0, g0, be0,
           w1, b1, g1, be1,
           w2, b2, g2, be2,
           wl, bl, gl, bel):
    B, C, N = xyz.shape
    _, D, _ = points.shape
    _, S, _ = nn_idx.shape
    out_ch = wl.shape[1]

    tbl = jnp.concatenate(
        [points.transpose(0, 2, 1), xyz.transpose(0, 2, 1),
         jnp.zeros((B, N, WIDTH - D - C), jnp.float32)], axis=-1
    ).reshape(B * N, WIDTH)
    flat_idx = (nn_idx.reshape(B, S * K).astype(jnp.int32)
                + (jnp.arange(B, dtype=jnp.int32) * N)[:, None]).reshape(-1)

    gathered = _sc_gather(tbl, flat_idx, B * S * K)

    nxt = jnp.repeat(new_xyz.transpose(0, 2, 1), K, axis=1)

    def center(wm, bv):
        n = wm.shape[1]
        cm = jnp.eye(n, dtype=jnp.float32) - jnp.full((n, n), 1.0 / n)
        return wm @ cm, bv @ cm

    w0c, b0c = center(w0, b0)
    w1c, b1c = center(w1, b1)
    w2c, b2c = center(w2, b2)
    wlr = wl.reshape(C + D, CMID, out_ch).transpose(1, 0, 2)
    wlp = jnp.zeros((CMID, CPAD, out_ch), jnp.float32).at[:, :C + D].set(wlr)
    c64 = jnp.eye(out_ch, dtype=jnp.float32) - jnp.full((out_ch, out_ch), 1.0 / out_ch)
    wlp = wlp @ c64
    blc = bl @ c64

    grid = (B, S // SBLK)
    out = pl.pallas_call(
        _tc_body,
        grid=grid,
        in_specs=[
            pl.BlockSpec((1, SBLK * K, WIDTH), lambda b, s: (b, s, 0)),
            pl.BlockSpec((1, SBLK * K, 3), lambda b, s: (b, s, 0)),
            pl.BlockSpec((C, 8), lambda b, s: (0, 0)),
            pl.BlockSpec((1, 8), lambda b, s: (0, 0)),
            pl.BlockSpec((1, 8), lambda b, s: (0, 0)),
            pl.BlockSpec((1, 8), lambda b, s: (0, 0)),
            pl.BlockSpec((8, 8), lambda b, s: (0, 0)),
            pl.BlockSpec((1, 8), lambda b, s: (0, 0)),
            pl.BlockSpec((1, 8), lambda b, s: (0, 0)),
            pl.BlockSpec((1, 8), lambda b, s: (0, 0)),
            pl.BlockSpec((8, CMID), lambda b, s: (0, 0)),
            pl.BlockSpec((1, CMID), lambda b, s: (0, 0)),
            pl.BlockSpec((1, CMID), lambda b, s: (0, 0)),
            pl.BlockSpec((1, CMID), lambda b, s: (0, 0)),
            pl.BlockSpec((CMID, CPAD, out_ch), lambda b, s: (0, 0, 0)),
            pl.BlockSpec((1, out_ch), lambda b, s: (0, 0)),
            pl.BlockSpec((1, out_ch), lambda b, s: (0, 0)),
            pl.BlockSpec((1, out_ch), lambda b, s: (0, 0)),
        ],
        out_specs=pl.BlockSpec((1, SBLK, out_ch), lambda b, s: (b, s, 0)),
        out_shape=jax.ShapeDtypeStruct((B, S, out_ch), jnp.float32),
    )(gathered.reshape(B, S * K, WIDTH), nxt,
      w0c, b0c.reshape(1, -1), g0.reshape(1, -1), be0.reshape(1, -1),
      w1c, b1c.reshape(1, -1), g1.reshape(1, -1), be1.reshape(1, -1),
      w2c, b2c.reshape(1, -1), g2.reshape(1, -1), be2.reshape(1, -1),
      wlp, blc.reshape(1, -1), gl.reshape(1, -1), bel.reshape(1, -1))

    return out.transpose(0, 2, 1)
